# Initial kernel scaffold; baseline (speedup 1.0000x reference)
import jax
import jax.numpy as jnp
from jax.experimental import pallas as pl


def kernel(x, adj_t, W1, b1, gamma, beta, W2, b2):
    raise NotImplementedError("write your pallas kernel here")


# speedup vs baseline: 16.2642x; 16.2642x over previous
import functools

import jax
import jax.numpy as jnp
from jax import lax
from jax.experimental import pallas as pl
from jax.experimental.pallas import tpu as pltpu
from jax.experimental.pallas import tpu_sc as plsc

N = 10000
E = 320000
D = 128

NC = 2
NS = 16
NW = NC * NS
EPT = E // NW
K = 80
NCHUNK = EPT // K
RPT = 640
RLAST = N - RPT * (NS - 1)

_mesh = plsc.VectorSubcoreMesh(core_axis_name="c", subcore_axis_name="s")


@functools.partial(
    pl.kernel,
    out_type=(
        jax.ShapeDtypeStruct((N,), jnp.float32),
        jax.ShapeDtypeStruct((N,), jnp.float32),
    ),
    mesh=_mesh,
    scratch_types=[
        pltpu.VMEM_SHARED((N,), jnp.float32),
        pltpu.VMEM((NCHUNK, K), jnp.int32),
        pltpu.VMEM((K,), jnp.float32),
        pltpu.VMEM((RPT,), jnp.float32),
    ],
)
def _deg_kernel(col_hbm, deg0_hbm, deg1_hbm, accum, cols_i, ones, zbuf):
    c = lax.axis_index("c")
    s = lax.axis_index("s")
    wid = c * NS + s

    for i in range(K // 16):
        ones[pl.ds(i * 16, 16)] = jnp.ones((16,), jnp.float32)

    def zfill(i, carry):
        zbuf[pl.ds(i * 16, 16)] = jnp.zeros((16,), jnp.float32)
        return carry

    lax.fori_loop(0, RPT // 16, zfill, 0)

    @pl.when(s < NS - 1)
    def _():
        pltpu.sync_copy(zbuf, accum.at[pl.ds(s * RPT, RPT)])

    @pl.when(s == NS - 1)
    def _():
        pltpu.sync_copy(zbuf.at[pl.ds(0, RLAST)], accum.at[pl.ds((NS - 1) * RPT, RLAST)])

    plsc.subcore_barrier()

    pltpu.sync_copy(col_hbm.at[wid], cols_i)

    def body(j, carry):
        pltpu.sync_copy(ones, accum.at[cols_i.at[j]], add=True)
        return carry

    lax.fori_loop(0, NCHUNK, body, 0)

    plsc.subcore_barrier()

    for core, dref in ((0, deg0_hbm), (1, deg1_hbm)):
        @pl.when(jnp.logical_and(c == core, s < NS - 1))
        def _(dref=dref):
            pltpu.sync_copy(accum.at[pl.ds(s * RPT, RPT)], zbuf)
            pltpu.sync_copy(zbuf, dref.at[pl.ds(s * RPT, RPT)])

        @pl.when(jnp.logical_and(c == core, s == NS - 1))
        def _(dref=dref):
            pltpu.sync_copy(accum.at[pl.ds((NS - 1) * RPT, RLAST)], zbuf.at[pl.ds(0, RLAST)])
            pltpu.sync_copy(zbuf.at[pl.ds(0, RLAST)], dref.at[pl.ds((NS - 1) * RPT, RLAST)])


@functools.partial(
    pl.kernel,
    out_type=jax.ShapeDtypeStruct((NC, N, D), jnp.float32),
    mesh=_mesh,
    scratch_types=[
        pltpu.VMEM_SHARED((N, D), jnp.float32),
        pltpu.VMEM((NCHUNK, K), jnp.int32),
        pltpu.VMEM((NCHUNK, K), jnp.int32),
        pltpu.VMEM((K, D), jnp.float32),
        pltpu.SemaphoreType.DMA,
    ],
)
def _agg_kernel(y_hbm, row_hbm, col_hbm, p_hbm, accum, rows_i, cols_i, buf0, sem0):
    c = lax.axis_index("c")
    s = lax.axis_index("s")
    wid = c * NS + s

    def zfill(i, carry):
        for j in range(D // 16):
            buf0[i, pl.ds(j * 16, 16)] = jnp.zeros((16,), jnp.float32)
        return carry

    lax.fori_loop(0, K, zfill, 0)

    nz = jnp.where(s < NS - 1, RPT // K, RLAST // K)

    def zb(k, carry):
        pltpu.sync_copy(buf0, accum.at[pl.ds(s * RPT + k * K, K)])
        return carry

    lax.fori_loop(0, nz, zb, 0)

    plsc.subcore_barrier()

    pltpu.sync_copy(row_hbm.at[wid], rows_i)
    pltpu.sync_copy(col_hbm.at[wid], cols_i)

    def body(j, carry):
        pltpu.async_copy(y_hbm.at[rows_i.at[j]], buf0, sem0).wait()
        pltpu.sync_copy(buf0, accum.at[cols_i.at[j]], add=True)
        return carry

    lax.fori_loop(0, NCHUNK, body, 0)

    plsc.subcore_barrier()

    def wb(k, carry):
        pltpu.sync_copy(accum.at[pl.ds(s * RPT + k * K, K)], buf0)
        pltpu.sync_copy(buf0, p_hbm.at[c, pl.ds(s * RPT + k * K, K)])
        return carry

    lax.fori_loop(0, nz, wb, 0)


_R = 1000


def _scale_matmul_body(x_ref, w1_ref, deg0_ref, deg1_ref, y1_ref, dis_ref):
    deg = deg0_ref[...] + deg1_ref[...]
    dis = jnp.where(deg > 0, lax.rsqrt(deg), 0.0)
    dis_ref[...] = dis
    xw = jnp.dot(x_ref[...], w1_ref[...], preferred_element_type=jnp.float32)
    y1_ref[...] = dis * xw


def _mid_body(p_ref, dis_ref, b1_ref, w2_ref, y2_ref):
    a = p_ref[0] + p_ref[1]
    dis = dis_ref[...]
    h = jnp.maximum(dis * a + b1_ref[...], 0.0)
    y2_ref[...] = dis * jnp.dot(h, w2_ref[...], preferred_element_type=jnp.float32)


def _final_body(p_ref, dis_ref, b2_ref, out_ref):
    out_ref[...] = dis_ref[...] * (p_ref[0] + p_ref[1]) + b2_ref[...]


def kernel(x, adj_t, W1, b1, gamma, beta, W2, b2):
    row = adj_t[0].astype(jnp.int32)
    col = adj_t[1].astype(jnp.int32)
    row3 = row.reshape(NW, NCHUNK, K)
    col3 = col.reshape(NW, NCHUNK, K)
    b1r = b1.reshape(1, D)
    b2r = b2.reshape(1, D)

    deg0, deg1 = _deg_kernel(col3)
    deg0 = deg0.reshape(N, 1)
    deg1 = deg1.reshape(N, 1)

    y1, dis = pl.pallas_call(
        _scale_matmul_body,
        grid=(N // _R,),
        in_specs=[
            pl.BlockSpec((_R, D), lambda i: (i, 0)),
            pl.BlockSpec((D, D), lambda i: (0, 0)),
            pl.BlockSpec((_R, 1), lambda i: (i, 0)),
            pl.BlockSpec((_R, 1), lambda i: (i, 0)),
        ],
        out_specs=[
            pl.BlockSpec((_R, D), lambda i: (i, 0)),
            pl.BlockSpec((_R, 1), lambda i: (i, 0)),
        ],
        out_shape=[
            jax.ShapeDtypeStruct((N, D), jnp.float32),
            jax.ShapeDtypeStruct((N, 1), jnp.float32),
        ],
    )(x, W1, deg0, deg1)

    p1 = _agg_kernel(y1, row3, col3)

    y2 = pl.pallas_call(
        _mid_body,
        grid=(N // _R,),
        in_specs=[
            pl.BlockSpec((NC, _R, D), lambda i: (0, i, 0)),
            pl.BlockSpec((_R, 1), lambda i: (i, 0)),
            pl.BlockSpec((1, D), lambda i: (0, 0)),
            pl.BlockSpec((D, D), lambda i: (0, 0)),
        ],
        out_specs=pl.BlockSpec((_R, D), lambda i: (i, 0)),
        out_shape=jax.ShapeDtypeStruct((N, D), jnp.float32),
    )(p1, dis, b1r, W2)

    p2 = _agg_kernel(y2, row3, col3)

    out = pl.pallas_call(
        _final_body,
        grid=(N // _R,),
        in_specs=[
            pl.BlockSpec((NC, _R, D), lambda i: (0, i, 0)),
            pl.BlockSpec((_R, 1), lambda i: (i, 0)),
            pl.BlockSpec((1, D), lambda i: (0, 0)),
        ],
        out_specs=pl.BlockSpec((_R, D), lambda i: (i, 0)),
        out_shape=jax.ShapeDtypeStruct((N, D), jnp.float32),
    )(p2, dis, b2r)

    return (out, out)


# speedup vs baseline: 24.9037x; 1.5312x over previous
import functools

import jax
import jax.numpy as jnp
from jax import lax
from jax.experimental import pallas as pl
from jax.experimental.pallas import tpu as pltpu
from jax.experimental.pallas import tpu_sc as plsc

N = 10000
E = 320000
D = 128

NC = 2
NS = 16
NW = NC * NS
EPT = E // NW
K = 80
NCHUNK = EPT // K
RPT = 640
RLAST = N - RPT * (NS - 1)

_mesh = plsc.VectorSubcoreMesh(core_axis_name="c", subcore_axis_name="s")


@functools.partial(
    pl.kernel,
    out_type=(
        jax.ShapeDtypeStruct((N,), jnp.float32),
        jax.ShapeDtypeStruct((N,), jnp.float32),
    ),
    mesh=_mesh,
    scratch_types=[
        pltpu.VMEM_SHARED((N,), jnp.float32),
        pltpu.VMEM((NCHUNK, K), jnp.int32),
        pltpu.VMEM((K,), jnp.float32),
        pltpu.VMEM((RPT,), jnp.float32),
    ],
)
def _deg_kernel(col_hbm, deg0_hbm, deg1_hbm, accum, cols_i, ones, zbuf):
    c = lax.axis_index("c")
    s = lax.axis_index("s")
    wid = c * NS + s

    for i in range(K // 16):
        ones[pl.ds(i * 16, 16)] = jnp.ones((16,), jnp.float32)

    def zfill(i, carry):
        zbuf[pl.ds(i * 16, 16)] = jnp.zeros((16,), jnp.float32)
        return carry

    lax.fori_loop(0, RPT // 16, zfill, 0)

    @pl.when(s < NS - 1)
    def _():
        pltpu.sync_copy(zbuf, accum.at[pl.ds(s * RPT, RPT)])

    @pl.when(s == NS - 1)
    def _():
        pltpu.sync_copy(zbuf.at[pl.ds(0, RLAST)], accum.at[pl.ds((NS - 1) * RPT, RLAST)])

    plsc.subcore_barrier()

    pltpu.sync_copy(col_hbm.at[wid], cols_i)

    def body(j, carry):
        pltpu.sync_copy(ones, accum.at[cols_i.at[j]], add=True)
        return carry

    lax.fori_loop(0, NCHUNK, body, 0)

    plsc.subcore_barrier()

    for core, dref in ((0, deg0_hbm), (1, deg1_hbm)):
        @pl.when(jnp.logical_and(c == core, s < NS - 1))
        def _(dref=dref):
            pltpu.sync_copy(accum.at[pl.ds(s * RPT, RPT)], zbuf)
            pltpu.sync_copy(zbuf, dref.at[pl.ds(s * RPT, RPT)])

        @pl.when(jnp.logical_and(c == core, s == NS - 1))
        def _(dref=dref):
            pltpu.sync_copy(accum.at[pl.ds((NS - 1) * RPT, RLAST)], zbuf.at[pl.ds(0, RLAST)])
            pltpu.sync_copy(zbuf.at[pl.ds(0, RLAST)], dref.at[pl.ds((NS - 1) * RPT, RLAST)])


@functools.partial(
    pl.kernel,
    out_type=jax.ShapeDtypeStruct((NC, N, D), jnp.float32),
    mesh=_mesh,
    scratch_types=[
        pltpu.VMEM_SHARED((N, D), jnp.float32),
        pltpu.VMEM((NCHUNK, K), jnp.int32),
        pltpu.VMEM((K,), jnp.int32),
        pltpu.VMEM((K,), jnp.int32),
        pltpu.VMEM((K, D), jnp.float32),
        pltpu.VMEM((K, D), jnp.float32),
        pltpu.SemaphoreType.DMA,
        pltpu.SemaphoreType.DMA,
        pltpu.SemaphoreType.DMA,
        pltpu.SemaphoreType.DMA,
    ],
)
def _agg_kernel(y_hbm, row_hbm, colf_hbm, p_hbm, accum, rows_i, colb0, colb1,
                buf0, buf1, gsem0, gsem1, csem0, csem1):
    c = lax.axis_index("c")
    s = lax.axis_index("s")
    wid = c * NS + s

    def zfill(i, carry):
        for j in range(D // 16):
            buf0[i, pl.ds(j * 16, 16)] = jnp.zeros((16,), jnp.float32)
        return carry

    lax.fori_loop(0, K, zfill, 0)

    nz = jnp.where(s < NS - 1, RPT // K, RLAST // K)

    def zb(k, carry):
        pltpu.sync_copy(buf0, accum.at[pl.ds(s * RPT + k * K, K)])
        return carry

    lax.fori_loop(0, nz, zb, 0)

    plsc.subcore_barrier()

    pltpu.sync_copy(row_hbm.at[wid], rows_i)

    ebase = wid * EPT
    pltpu.async_copy(colf_hbm.at[pl.ds(ebase, K)], colb0, csem0)
    pltpu.async_copy(colf_hbm.at[pl.ds(ebase + K, K)], colb1, csem1)
    pltpu.async_copy(y_hbm.at[rows_i.at[0]], buf0, gsem0)
    pltpu.async_copy(y_hbm.at[rows_i.at[1]], buf1, gsem1)

    def body(i, carry):
        j = 2 * i
        pltpu.make_async_copy(y_hbm.at[rows_i.at[j]], buf0, gsem0).wait()
        pltpu.make_async_copy(colf_hbm.at[pl.ds(ebase + j * K, K)], colb0, csem0).wait()
        pltpu.sync_copy(buf0, accum.at[colb0], add=True)
        nxt = jnp.minimum(j + 2, NCHUNK - 1)
        pltpu.async_copy(colf_hbm.at[pl.ds(ebase + nxt * K, K)], colb0, csem0)
        pltpu.async_copy(y_hbm.at[rows_i.at[nxt]], buf0, gsem0)
        pltpu.make_async_copy(y_hbm.at[rows_i.at[j + 1]], buf1, gsem1).wait()
        pltpu.make_async_copy(colf_hbm.at[pl.ds(ebase + (j + 1) * K, K)], colb1, csem1).wait()
        pltpu.sync_copy(buf1, accum.at[colb1], add=True)
        nxt1 = jnp.minimum(j + 3, NCHUNK - 1)
        pltpu.async_copy(colf_hbm.at[pl.ds(ebase + nxt1 * K, K)], colb1, csem1)
        pltpu.async_copy(y_hbm.at[rows_i.at[nxt1]], buf1, gsem1)
        return carry

    lax.fori_loop(0, (NCHUNK - 1) // 2, body, 0)
    ltail = pl.ds(ebase + (NCHUNK - 1) * K, K)
    pltpu.make_async_copy(y_hbm.at[rows_i.at[NCHUNK - 1]], buf1, gsem1).wait()
    pltpu.make_async_copy(colf_hbm.at[ltail], colb1, csem1).wait()
    pltpu.make_async_copy(y_hbm.at[rows_i.at[NCHUNK - 1]], buf0, gsem0).wait()
    pltpu.make_async_copy(colf_hbm.at[ltail], colb0, csem0).wait()
    pltpu.sync_copy(buf0, accum.at[colb0], add=True)

    plsc.subcore_barrier()

    def wb(k, carry):
        pltpu.sync_copy(accum.at[pl.ds(s * RPT + k * K, K)], buf0)
        pltpu.sync_copy(buf0, p_hbm.at[c, pl.ds(s * RPT + k * K, K)])
        return carry

    lax.fori_loop(0, nz, wb, 0)


_R = 1000


def _scale_matmul_body(x_ref, w1_ref, deg0_ref, deg1_ref, y1_ref, dis_ref):
    deg = deg0_ref[...] + deg1_ref[...]
    dis = jnp.where(deg > 0, lax.rsqrt(deg), 0.0)
    dis_ref[...] = dis
    xw = jnp.dot(x_ref[...], w1_ref[...], preferred_element_type=jnp.float32)
    y1_ref[...] = dis * xw


def _mid_body(p_ref, dis_ref, b1_ref, w2_ref, y2_ref):
    a = p_ref[0] + p_ref[1]
    dis = dis_ref[...]
    h = jnp.maximum(dis * a + b1_ref[...], 0.0)
    y2_ref[...] = dis * jnp.dot(h, w2_ref[...], preferred_element_type=jnp.float32)


def _final_body(p_ref, dis_ref, b2_ref, out_ref):
    out_ref[...] = dis_ref[...] * (p_ref[0] + p_ref[1]) + b2_ref[...]


def kernel(x, adj_t, W1, b1, gamma, beta, W2, b2):
    row = adj_t[0].astype(jnp.int32)
    col = adj_t[1].astype(jnp.int32)
    row3 = row.reshape(NW, NCHUNK, K)
    col3 = col.reshape(NW, NCHUNK, K)
    b1r = b1.reshape(1, D)
    b2r = b2.reshape(1, D)

    deg0, deg1 = _deg_kernel(col3)
    deg0 = deg0.reshape(N, 1)
    deg1 = deg1.reshape(N, 1)

    y1, dis = pl.pallas_call(
        _scale_matmul_body,
        grid=(N // _R,),
        in_specs=[
            pl.BlockSpec((_R, D), lambda i: (i, 0)),
            pl.BlockSpec((D, D), lambda i: (0, 0)),
            pl.BlockSpec((_R, 1), lambda i: (i, 0)),
            pl.BlockSpec((_R, 1), lambda i: (i, 0)),
        ],
        out_specs=[
            pl.BlockSpec((_R, D), lambda i: (i, 0)),
            pl.BlockSpec((_R, 1), lambda i: (i, 0)),
        ],
        out_shape=[
            jax.ShapeDtypeStruct((N, D), jnp.float32),
            jax.ShapeDtypeStruct((N, 1), jnp.float32),
        ],
    )(x, W1, deg0, deg1)

    p1 = _agg_kernel(y1, row3, col)

    y2 = pl.pallas_call(
        _mid_body,
        grid=(N // _R,),
        in_specs=[
            pl.BlockSpec((NC, _R, D), lambda i: (0, i, 0)),
            pl.BlockSpec((_R, 1), lambda i: (i, 0)),
            pl.BlockSpec((1, D), lambda i: (0, 0)),
            pl.BlockSpec((D, D), lambda i: (0, 0)),
        ],
        out_specs=pl.BlockSpec((_R, D), lambda i: (i, 0)),
        out_shape=jax.ShapeDtypeStruct((N, D), jnp.float32),
    )(p1, dis, b1r, W2)

    p2 = _agg_kernel(y2, row3, col)

    out = pl.pallas_call(
        _final_body,
        grid=(N // _R,),
        in_specs=[
            pl.BlockSpec((NC, _R, D), lambda i: (0, i, 0)),
            pl.BlockSpec((_R, 1), lambda i: (i, 0)),
            pl.BlockSpec((1, D), lambda i: (0, 0)),
        ],
        out_specs=pl.BlockSpec((_R, D), lambda i: (i, 0)),
        out_shape=jax.ShapeDtypeStruct((N, D), jnp.float32),
    )(p2, dis, b2r)

    return (out, out)
